# baseline (speedup 1.0000x reference)
import numpy as np

import jax
import jax.numpy as jnp
from jax import lax
from jax.experimental import pallas as pl
from jax.experimental.pallas import tpu as pltpu
from jax.experimental.pallas import tpu_sc as plsc

NUM_USER = 5000
NUM_ITEM = 5000
DIM = 128
N_NODES = NUM_USER + NUM_ITEM
N_EDGES = 320000

NC = 2
NS = 16
NW = NC * NS
LANES = 16

CHUNK = 16
K_PER_W = 640
EDGES_PER_W = CHUNK * K_PER_W
E_PAD = NW * EDGES_PER_W
N_PAD = 10112
ROWS_PER_TILE = N_PAD // NS
N_TAB = N_PAD // 2
N_TAB_SP = 5008

NGB = 2
NSB = 2
NIX = 8
IDEPTH = 6

_UNPACK_PERM = np.concatenate(
    [np.concatenate([np.arange(0, 32, 2), np.arange(1, 32, 2)]) + 32 * j
     for j in range(DIM // 32)])


def _scale_unpack_chunk(gbuf, sbuf, wvec, pv):
    def _rows(i4, _):
        for di in range(4):
            i = i4 * 4 + di
            idx = jnp.full((LANES,), i, jnp.int32)
            wb = wvec.at[idx].get(mode="promise_in_bounds")
            m = jnp.int32(0) - pv.at[idx].get(mode="promise_in_bounds")
            for j in range(DIM // 32):
                lo = gbuf[i, pl.ds(LANES * j, LANES)]
                hi = gbuf[i, pl.ds(64 + LANES * j, LANES)]
                v = (hi & m) | (lo & ~m)
                a = lax.bitcast_convert_type(v << 16, jnp.float32)
                b = lax.bitcast_convert_type(v & jnp.int32(-65536),
                                             jnp.float32)
                sbuf[i, pl.ds(32 * j, LANES)] = a * wb
                sbuf[i, pl.ds(32 * j + LANES, LANES)] = b * wb
        return 0
    lax.fori_loop(0, CHUNK // 4, _rows, 0)


def _spmm_body(xb_hbm, col_hbm, row_hbm, w_hbm, out_hbm, acc, table, *rest):
    gb = rest[0:NGB]
    sb = rest[NGB:NGB + NSB]
    ibA, cbA, wbA = rest[NGB + NSB:NGB + NSB + 3]
    o0 = NGB + NSB + 3
    rb = rest[o0:o0 + NIX]
    o1 = o0 + NIX
    gsem = rest[o1:o1 + NGB]
    ssem = rest[o1 + NGB:o1 + NGB + NSB]
    isem = rest[o1 + NGB + NSB:o1 + NGB + NSB + NIX]

    c = lax.axis_index("c")
    s = lax.axis_index("s")
    wid = s * NC + c
    base = wid * EDGES_PER_W

    def _issue_idx(k, t):
        eoff = base + k * CHUNK
        pltpu.async_copy(col_hbm.at[pl.ds(eoff, CHUNK)],
                         cbA.at[pl.ds(t * CHUNK, CHUNK)], isem[t])
        pltpu.async_copy(row_hbm.at[pl.ds(eoff, CHUNK)], rb[t], isem[t])
        pltpu.async_copy(w_hbm.at[pl.ds(eoff, CHUNK)],
                         wbA.at[pl.ds(t * CHUNK, CHUNK)], isem[t])

    def _wait_idx(t):
        pltpu.make_async_copy(col_hbm.at[pl.ds(0, CHUNK)],
                              cbA.at[pl.ds(t * CHUNK, CHUNK)], isem[t]).wait()
        pltpu.make_async_copy(row_hbm.at[pl.ds(0, CHUNK)], rb[t], isem[t]).wait()
        pltpu.make_async_copy(w_hbm.at[pl.ds(0, CHUNK)],
                              wbA.at[pl.ds(t * CHUNK, CHUNK)], isem[t]).wait()

    def _issue_gather(t, p):
        ibA[pl.ds(p * CHUNK, CHUNK)] = cbA[pl.ds(t * CHUNK, CHUNK)] >> 1
        pltpu.async_copy(table.at[ibA.at[pl.ds(p * CHUNK, CHUNK)]],
                         gb[p], gsem[p])

    def _wait_gather(p):
        pltpu.make_async_copy(table.at[ibA.at[pl.ds(0, CHUNK)]],
                              gb[p], gsem[p]).wait()

    def _issue_scatter(p, t):
        pltpu.async_copy(sb[p], acc.at[rb[t]], ssem[p], add=True)

    def _wait_scatter(p):
        pltpu.make_async_copy(sb[p], acc.at[rb[0]], ssem[p]).wait()

    r0 = s * ROWS_PER_TILE

    @pl.when(s < NS - 1)
    def _():
        pltpu.sync_copy(xb_hbm.at[pl.ds(s * 320, 320)],
                        table.at[pl.ds(s * 320, 320)])

    @pl.when(s == NS - 1)
    def _():
        pltpu.sync_copy(xb_hbm.at[pl.ds(4800, N_TAB_SP - 4800)],
                        table.at[pl.ds(4800, N_TAB_SP - 4800)])

    def _zero_buf(i, _):
        for j in range(DIM // LANES):
            sb[0][i, pl.ds(LANES * j, LANES)] = jnp.zeros((LANES,), jnp.float32)
        return 0
    lax.fori_loop(0, CHUNK, _zero_buf, 0)

    def _zero_acc(k, _):
        pltpu.sync_copy(sb[0], acc.at[pl.ds(r0 + k * CHUNK, CHUNK)])
        return 0
    lax.fori_loop(0, ROWS_PER_TILE // CHUNK, _zero_acc, 0)
    pltpu.sync_copy(sb[0].at[pl.ds(0, ROWS_PER_TILE % CHUNK)],
                    acc.at[pl.ds(r0 + ROWS_PER_TILE - ROWS_PER_TILE % CHUNK,
                                 ROWS_PER_TILE % CHUNK)])
    plsc.subcore_barrier()

    for t in range(IDEPTH):
        _issue_idx(t, t)
    _wait_idx(0)
    _issue_gather(0, 0)

    def _octet(k8, _):
        for u in range(NIX):
            k = NIX * k8 + u
            if u >= 2:
                _wait_scatter(u % NSB)
            else:
                @pl.when(k8 > 0)
                def _():
                    _wait_scatter(u % NSB)
            _wait_idx((u + 1) % NIX)
            _issue_gather((u + 1) % NIX, (u + 1) % NGB)
            kpi = jnp.minimum(k + IDEPTH, K_PER_W - 1)
            _issue_idx(kpi, (u + IDEPTH) % NIX)
            _wait_gather(u % NGB)
            _scale_unpack_chunk(
                gb[u % NGB], sb[u % NSB],
                wbA[pl.ds(u * CHUNK, CHUNK)],
                cbA[pl.ds(u * CHUNK, CHUNK)] & 1)
            _issue_scatter(u % NSB, u)
        return 0
    lax.fori_loop(0, K_PER_W // NIX, _octet, 0)

    _wait_gather(K_PER_W % NGB)
    for t in range(1, IDEPTH):
        _wait_idx((K_PER_W + t) % NIX)
    _wait_scatter((K_PER_W - 2) % NSB)
    _wait_scatter((K_PER_W - 1) % NSB)
    plsc.subcore_barrier()

    pltpu.sync_copy(acc.at[pl.ds(r0, ROWS_PER_TILE)],
                    out_hbm.at[c, pl.ds(r0, ROWS_PER_TILE)])


def _spmm_sc(xb, col, row, w):
    mesh = plsc.VectorSubcoreMesh(core_axis_name="c", subcore_axis_name="s")
    scratch = (
        [pltpu.VMEM_SHARED((N_PAD, DIM), jnp.float32)]
        + [pltpu.VMEM_SHARED((N_TAB_SP, DIM), jnp.int32)]
        + [pltpu.VMEM((CHUNK, DIM), jnp.int32)] * NGB
        + [pltpu.VMEM((CHUNK, DIM), jnp.float32)] * NSB
        + [pltpu.VMEM((NGB * CHUNK,), jnp.int32)]
        + [pltpu.VMEM((NIX * CHUNK,), jnp.int32)]
        + [pltpu.VMEM((NIX * CHUNK,), jnp.float32)]
        + [pltpu.VMEM((CHUNK,), jnp.int32)] * NIX
        + [pltpu.SemaphoreType.DMA] * (NGB + NSB + NIX)
    )
    return pl.kernel(
        _spmm_body,
        out_type=jax.ShapeDtypeStruct((NC, N_PAD, DIM), jnp.float32),
        mesh=mesh,
        scratch_types=scratch,
    )(xb, col, row, w)


def _lrelu(v):
    return jnp.where(v > 0, v, 0.1 * v)


def _dense_body(p_ref, w1_ref, w3_ref, w4_ref, o_ref, ob_ref):
    p = p_ref[0] + p_ref[1]
    y = _lrelu(p)
    nt = (((1,), (1,)), ((), ()))
    h = lax.dot_general(y, w1_ref[...], nt, preferred_element_type=jnp.float32)
    g = _lrelu(lax.dot_general(h, w3_ref[...], nt,
                               preferred_element_type=jnp.float32))
    x = lax.dot_general(g, w4_ref[...], nt, preferred_element_type=jnp.float32)
    o_ref[...] = x
    ob_ref[...] = x.astype(jnp.bfloat16)


def _dense_tc(partials, W_h1p, W_3, W_4):
    nblk = 8
    rbk = N_PAD // nblk
    return pl.pallas_call(
        _dense_body,
        grid=(nblk,),
        in_specs=[
            pl.BlockSpec((NC, rbk, DIM), lambda i: (0, i, 0)),
            pl.BlockSpec((DIM, DIM), lambda i: (0, 0)),
            pl.BlockSpec((DIM, DIM), lambda i: (0, 0)),
            pl.BlockSpec((DIM, DIM), lambda i: (0, 0)),
        ],
        out_specs=[
            pl.BlockSpec((rbk, DIM), lambda i: (i, 0)),
            pl.BlockSpec((rbk, DIM), lambda i: (i, 0)),
        ],
        out_shape=[
            jax.ShapeDtypeStruct((N_NODES, DIM), jnp.float32),
            jax.ShapeDtypeStruct((N_PAD, DIM), jnp.bfloat16),
        ],
    )(partials, W_h1p, W_3, W_4)


def _pack_pairs(xb):
    return lax.bitcast_convert_type(
        xb.reshape(-1, 2), jnp.int32).reshape(N_TAB, DIM)


def kernel(edge_index, edge_weight, user_emb, item_emb, W_h1, W_3, W_4):
    x0 = jnp.concatenate([user_emb, item_emb], axis=0)
    x0b = jnp.concatenate(
        [x0, jnp.zeros((N_PAD - N_NODES, DIM), jnp.float32)],
        axis=0).astype(jnp.bfloat16)
    W_h1p = W_h1[:, _UNPACK_PERM]

    pad = E_PAD - N_EDGES
    row = jnp.concatenate([edge_index[0], jnp.zeros((pad,), jnp.int32)])
    col = jnp.concatenate([edge_index[1], jnp.zeros((pad,), jnp.int32)])
    w = jnp.concatenate([edge_weight, jnp.zeros((pad,), jnp.float32)])

    p1 = _spmm_sc(_pack_pairs(x0b), col, row, w)
    x1, x1b = _dense_tc(p1, W_h1p, W_3, W_4)
    p2 = _spmm_sc(_pack_pairs(x1b), col, row, w)
    x2, _ = _dense_tc(p2, W_h1p, W_3, W_4)

    return (x0, x1, x2, user_emb, item_emb, W_h1, W_3, W_4)


# speedup vs baseline: 1.9309x; 1.9309x over previous
import jax
import jax.numpy as jnp
from jax import lax
from jax.experimental import pallas as pl
from jax.experimental.pallas import tpu as pltpu
from jax.experimental.pallas import tpu_sc as plsc

NUM_USER = 5000
NUM_ITEM = 5000
DIM = 128
N_NODES = NUM_USER + NUM_ITEM
N_EDGES = 320000

NC = 2
NS = 16
NW = NC * NS
LANES = 16

CHUNK = 32
K_PER_W = 320
EDGES_PER_W = CHUNK * K_PER_W
E_PAD = NW * EDGES_PER_W
N_PAD = 10240
ROWS_PER_TILE = N_PAD // NS

NGB = 8
NIX = 8
GDEPTH = 4
IDEPTH = 6


def _scale_chunk(gbuf, wbuf):
    def _group(g, _):
        wvec = wbuf[pl.ds(g * LANES, LANES)]

        def _rows(i4, _):
            for di in range(4):
                i = i4 * 4 + di
                wb = wvec.at[jnp.full((LANES,), i, jnp.int32)].get(
                    mode="promise_in_bounds")
                r = g * LANES + i
                for j in range(DIM // LANES):
                    sl = (r, pl.ds(LANES * j, LANES))
                    gbuf[sl] = gbuf[sl] * wb
            return 0
        lax.fori_loop(0, LANES // 4, _rows, 0)
        return 0
    lax.fori_loop(0, CHUNK // LANES, _group, 0)


def _spmm_body(x_hbm, col_hbm, row_hbm, w_hbm, out_hbm, acc, *rest):
    gb = rest[0:NGB]
    cb = rest[NGB:NGB + NIX]
    rb = rest[NGB + NIX:NGB + 2 * NIX]
    wb = rest[NGB + 2 * NIX:NGB + 3 * NIX]
    gsem = rest[NGB + 3 * NIX:2 * NGB + 3 * NIX]
    ssem = rest[2 * NGB + 3 * NIX:3 * NGB + 3 * NIX]
    isem = rest[3 * NGB + 3 * NIX:3 * NGB + 4 * NIX]

    c = lax.axis_index("c")
    s = lax.axis_index("s")
    wid = s * NC + c
    base = wid * EDGES_PER_W

    def _issue_idx(k, t):
        eoff = base + k * CHUNK
        pltpu.async_copy(col_hbm.at[pl.ds(eoff, CHUNK)], cb[t], isem[t])
        pltpu.async_copy(row_hbm.at[pl.ds(eoff, CHUNK)], rb[t], isem[t])
        pltpu.async_copy(w_hbm.at[pl.ds(eoff, CHUNK)], wb[t], isem[t])

    def _wait_idx(t):
        pltpu.make_async_copy(col_hbm.at[pl.ds(0, CHUNK)], cb[t], isem[t]).wait()
        pltpu.make_async_copy(row_hbm.at[pl.ds(0, CHUNK)], rb[t], isem[t]).wait()
        pltpu.make_async_copy(w_hbm.at[pl.ds(0, CHUNK)], wb[t], isem[t]).wait()

    def _issue_gather(t, p):
        pltpu.async_copy(x_hbm.at[cb[t]], gb[p], gsem[p])

    def _wait_gather(p):
        pltpu.make_async_copy(x_hbm.at[cb[0]], gb[p], gsem[p]).wait()

    def _issue_scatter(p, t):
        pltpu.async_copy(gb[p], acc.at[rb[t]], ssem[p], add=True)

    def _wait_scatter(p):
        pltpu.make_async_copy(gb[p], acc.at[rb[0]], ssem[p]).wait()

    def _zero_buf(i, _):
        for j in range(DIM // LANES):
            gb[0][i, pl.ds(LANES * j, LANES)] = jnp.zeros((LANES,), jnp.float32)
        return 0
    lax.fori_loop(0, CHUNK, _zero_buf, 0)

    def _zero_acc(k, _):
        pltpu.sync_copy(gb[0],
                        acc.at[pl.ds(s * ROWS_PER_TILE + k * CHUNK, CHUNK)])
        return 0
    lax.fori_loop(0, ROWS_PER_TILE // CHUNK, _zero_acc, 0)
    plsc.subcore_barrier()

    for t in range(IDEPTH):
        _issue_idx(t, t)
    for t in range(GDEPTH):
        _wait_idx(t)
        _issue_gather(t, t)

    def _octet(k8, _):
        for u in range(NIX):
            k = NIX * k8 + u
            p = u % NGB
            pg = (u + GDEPTH) % NGB
            qg = (u + GDEPTH) % NIX
            qi = (u + IDEPTH) % NIX
            if u >= 2:
                _wait_scatter((u - 2) % NGB)
            else:
                @pl.when(k8 > 0)
                def _():
                    _wait_scatter((u - 2) % NGB)
            _wait_idx(qg)
            _issue_gather(qg, pg)
            kpi = jnp.minimum(k + IDEPTH, K_PER_W - 1)
            _issue_idx(kpi, qi)
            _wait_gather(p)
            _scale_chunk(gb[p], wb[u])
            _issue_scatter(p, u)
        return 0
    lax.fori_loop(0, K_PER_W // NIX, _octet, 0)

    for t in range(GDEPTH):
        _wait_gather(t % NGB)
    for t in range(GDEPTH, IDEPTH):
        _wait_idx(t % NIX)
    _wait_scatter((K_PER_W - 2) % NGB)
    _wait_scatter((K_PER_W - 1) % NGB)
    plsc.subcore_barrier()

    r0 = s * ROWS_PER_TILE
    pltpu.sync_copy(acc.at[pl.ds(r0, ROWS_PER_TILE)],
                    out_hbm.at[c, pl.ds(r0, ROWS_PER_TILE)])


def _spmm_sc(x, col, row, w):
    mesh = plsc.VectorSubcoreMesh(core_axis_name="c", subcore_axis_name="s")
    scratch = (
        [pltpu.VMEM_SHARED((N_PAD, DIM), jnp.float32)]
        + [pltpu.VMEM((CHUNK, DIM), jnp.float32)] * NGB
        + [pltpu.VMEM((CHUNK,), jnp.int32)] * NIX
        + [pltpu.VMEM((CHUNK,), jnp.int32)] * NIX
        + [pltpu.VMEM((CHUNK,), jnp.float32)] * NIX
        + [pltpu.SemaphoreType.DMA] * (2 * NGB + NIX)
    )
    return pl.kernel(
        _spmm_body,
        out_type=jax.ShapeDtypeStruct((NC, N_PAD, DIM), jnp.float32),
        mesh=mesh,
        scratch_types=scratch,
    )(x, col, row, w)


def _lrelu(v):
    return jnp.where(v > 0, v, 0.1 * v)


def _dense_body(p_ref, w1_ref, w3_ref, w4_ref, o_ref):
    p = p_ref[0] + p_ref[1]
    y = _lrelu(p)
    nt = (((1,), (1,)), ((), ()))
    h = lax.dot_general(y, w1_ref[...], nt, preferred_element_type=jnp.float32)
    g = _lrelu(lax.dot_general(h, w3_ref[...], nt,
                               preferred_element_type=jnp.float32))
    o_ref[...] = lax.dot_general(g, w4_ref[...], nt,
                                 preferred_element_type=jnp.float32)


def _dense_tc(partials, W_h1, W_3, W_4):
    nblk = 8
    rbk = N_PAD // nblk
    return pl.pallas_call(
        _dense_body,
        grid=(nblk,),
        in_specs=[
            pl.BlockSpec((NC, rbk, DIM), lambda i: (0, i, 0)),
            pl.BlockSpec((DIM, DIM), lambda i: (0, 0)),
            pl.BlockSpec((DIM, DIM), lambda i: (0, 0)),
            pl.BlockSpec((DIM, DIM), lambda i: (0, 0)),
        ],
        out_specs=pl.BlockSpec((rbk, DIM), lambda i: (i, 0)),
        out_shape=jax.ShapeDtypeStruct((N_NODES, DIM), jnp.float32),
    )(partials, W_h1, W_3, W_4)


def kernel(edge_index, edge_weight, user_emb, item_emb, W_h1, W_3, W_4):
    x0 = jnp.concatenate([user_emb, item_emb], axis=0)

    pad = E_PAD - N_EDGES
    row = jnp.concatenate([edge_index[0], jnp.zeros((pad,), jnp.int32)])
    col = jnp.concatenate([edge_index[1], jnp.zeros((pad,), jnp.int32)])
    w = jnp.concatenate([edge_weight, jnp.zeros((pad,), jnp.float32)])

    p1 = _spmm_sc(x0, col, row, w)
    x1 = _dense_tc(p1, W_h1, W_3, W_4)
    p2 = _spmm_sc(x1, col, row, w)
    x2 = _dense_tc(p2, W_h1, W_3, W_4)

    return (x0, x1, x2, user_emb, item_emb, W_h1, W_3, W_4)
